# baseline (speedup 1.0000x reference)
import functools

import jax
import jax.numpy as jnp
from jax import lax
from jax.experimental import pallas as pl
from jax.experimental.pallas import tpu as pltpu
from jax.experimental.pallas import tpu_sc as plsc

_NT = 25


def _sc_call(xt):
    info = plsc.get_sparse_core_info()
    nc = info.num_cores
    mesh = plsc.VectorSubcoreMesh(core_axis_name="c", subcore_axis_name="s")

    @functools.partial(
        pl.kernel,
        mesh=mesh,
        out_type=jax.ShapeDtypeStruct((100, 64, 4096), jnp.float32),
        scratch_types=[
            pltpu.VMEM((8, 4096), jnp.float32),
            pltpu.VMEM((8, 4096), jnp.float32),
            pltpu.VMEM((8, 4096), jnp.float32),
            pltpu.SemaphoreType.DMA,
            pltpu.SemaphoreType.DMA,
            pltpu.SemaphoreType.DMA,
            pltpu.SemaphoreType.DMA,
        ],
    )
    def sc(xt_hbm, out_hbm, off_v, w0, w1, si0, si1, so0, so1):
        wid = lax.axis_index("s") * nc + lax.axis_index("c")
        jj8 = (wid % 8) * 8
        krem = wid // 8

        pltpu.sync_copy(xt_hbm.at[0, pl.ds(jj8, 8), :], off_v)

        bufs = (w0, w1)
        sin = (si0, si1)
        sout = (so0, so1)

        def _make_subtract(w_v):
            def subtract(i, carry):
                r = i >> 6
                cb = (i & 63) * 64
                for u in range(4):
                    sl = pl.ds(cb + u * 16, 16)
                    w_v[r, sl] = w_v[r, sl] - off_v[r, sl]
                return carry

            return subtract

        subs = (_make_subtract(w0), _make_subtract(w1))

        def start_in(t):
            k = krem + 4 * t
            return pltpu.async_copy(
                xt_hbm.at[50 + k, pl.ds(jj8, 8), :], bufs[t % 2], sin[t % 2]
            )

        def start_out(t):
            k = krem + 4 * t
            return pltpu.async_copy(
                bufs[t % 2], out_hbm.at[k, pl.ds(jj8, 8), :], sout[t % 2]
            )

        in_h = {0: start_in(0)}
        out_h = {}
        for t in range(_NT):
            in_h.pop(t).wait()
            if t >= 1:
                out_h.pop(t - 1).wait()
            if t + 1 < _NT:
                in_h[t + 1] = start_in(t + 1)
            lax.fori_loop(0, 512, subs[t % 2], 0)
            out_h[t] = start_out(t)
        out_h.pop(_NT - 1).wait()

    return sc(xt)


def kernel(x, W):
    del W
    xt = jnp.transpose(x, (1, 2, 0))
    out_t = _sc_call(xt)
    return jnp.transpose(out_t, (2, 0, 1))
